# baseline (speedup 1.0000x reference)
import functools
import jax
import jax.numpy as jnp
from jax import lax
from jax.experimental import pallas as pl
from jax.experimental.pallas import tpu as pltpu
from jax.experimental.pallas import tpu_sc as plsc

B, N, L = 1, 128, 384
D_INIT, D_MSA, D_PAIR, D_STATE = 48, 256, 128, 32
NBIN = 65
NSEQ = 22

_NW = 32
_ROWS_PER_W = L // _NW


def _sc_body(seq_hbm, idx_hbm, left_hbm, right_hbm, pos_hbm, sttbl_hbm,
             pair_out, state_out,
             seq_v, idx_v, left_v, right_v, pos_v, sttbl_v,
             rowbuf0, rowbuf1, stbuf, sem0, sem1):
    cid = lax.axis_index("c")
    sid = lax.axis_index("s")
    w = sid * 2 + cid
    base = w * _ROWS_PER_W

    pltpu.sync_copy(seq_hbm, seq_v)
    pltpu.sync_copy(idx_hbm, idx_v)
    pltpu.sync_copy(left_hbm, left_v)
    pltpu.sync_copy(right_hbm, right_v)
    pltpu.sync_copy(pos_hbm, pos_v)
    pltpu.sync_copy(sttbl_hbm, sttbl_v)

    iota = lax.iota(jnp.int32, 16)

    @pl.when(w < L // 16)
    def _():
        sbase = w * 16
        for jj in range(16):
            j16 = jnp.full((16,), sbase + jj, jnp.int32)
            sj = plsc.load_gather(seq_v, [j16])
            for cc in range(D_STATE // 16):
                stbuf[jj, pl.ds(cc * 16, 16)] = plsc.load_gather(
                    sttbl_v, [sj, iota + cc * 16])
        pltpu.sync_copy(stbuf, state_out.at[pl.ds(sbase, 16)])

    bufs = (rowbuf0, rowbuf1)
    sems = (sem0, sem1)
    pending = [None, None]
    for rr in range(_ROWS_PER_W):
        i = base + rr
        k = rr % 2
        if pending[k] is not None:
            pending[k].wait()
        buf = bufs[k]
        i16 = jnp.full((16,), i, jnp.int32)
        si = plsc.load_gather(seq_v, [i16])
        di = plsc.load_gather(idx_v, [i16])
        rrow = [plsc.load_gather(right_v, [si, iota + 16 * cc])
                for cc in range(D_PAIR // 16)]

        @plsc.parallel_loop(0, L, unroll=4)
        def jbody(j, buf=buf, di=di, rrow=rrow):
            j16 = jnp.full((16,), j, jnp.int32)
            sj = plsc.load_gather(seq_v, [j16])
            dj = plsc.load_gather(idx_v, [j16])
            pidx = jnp.clip(dj - di + 32, 0, NBIN - 1)
            for cc in range(D_PAIR // 16):
                lv = plsc.load_gather(left_v, [sj, iota + 16 * cc])
                pv = plsc.load_gather(pos_v, [pidx, iota + 16 * cc])
                buf[j, pl.ds(16 * cc, 16)] = lv + pv + rrow[cc]

        pending[k] = pltpu.async_copy(buf, pair_out.at[i], sems[k])
    pending[0].wait()
    pending[1].wait()


def _sc_pair_state(seq, idx, emb_left, emb_right, pos_emb, emb_state):
    mesh = plsc.VectorSubcoreMesh(core_axis_name="c", subcore_axis_name="s")
    kern = pl.kernel(
        _sc_body,
        out_type=[
            jax.ShapeDtypeStruct((L, L, D_PAIR), jnp.float32),
            jax.ShapeDtypeStruct((L, D_STATE), jnp.float32),
        ],
        mesh=mesh,
        compiler_params=pltpu.CompilerParams(needs_layout_passes=False),
        scratch_types=[
            pltpu.VMEM((L,), jnp.int32),
            pltpu.VMEM((L,), jnp.int32),
            pltpu.VMEM((NSEQ, D_PAIR), jnp.float32),
            pltpu.VMEM((NSEQ, D_PAIR), jnp.float32),
            pltpu.VMEM((NBIN, D_PAIR), jnp.float32),
            pltpu.VMEM((NSEQ, D_STATE), jnp.float32),
            pltpu.VMEM((L, D_PAIR), jnp.float32),
            pltpu.VMEM((L, D_PAIR), jnp.float32),
            pltpu.VMEM((16, D_STATE), jnp.float32),
            pltpu.SemaphoreType.DMA,
            pltpu.SemaphoreType.DMA,
        ],
    )
    return kern(seq, idx, emb_left, emb_right, pos_emb, emb_state)


_N_BLK = 8


def _tc_body(seq_ref, msa_ref, w_ref, b_ref, q_ref, out_ref, qrow):
    n = pl.program_id(0)

    @pl.when(n == 0)
    def _():
        seq = seq_ref[...]
        onehot = (seq == lax.broadcasted_iota(jnp.int32, (L, NSEQ), 1)
                  ).astype(jnp.float32)
        qrow[...] = (jnp.dot(onehot, q_ref[...],
                             preferred_element_type=jnp.float32)
                     + b_ref[...])

    x = msa_ref[...]
    y = lax.dot_general(x, w_ref[...], (((2,), (1,)), ((), ())),
                        preferred_element_type=jnp.float32)
    out_ref[...] = y + qrow[...][None]


def _tc_msa(seq2d, msa3, emb_W, emb_b, emb_q):
    grid = (N // _N_BLK,)
    return pl.pallas_call(
        _tc_body,
        grid=grid,
        in_specs=[
            pl.BlockSpec((L, 1), lambda n: (0, 0)),
            pl.BlockSpec((_N_BLK, L, D_INIT), lambda n: (n, 0, 0)),
            pl.BlockSpec((D_MSA, D_INIT), lambda n: (0, 0)),
            pl.BlockSpec((1, D_MSA), lambda n: (0, 0)),
            pl.BlockSpec((NSEQ, D_MSA), lambda n: (0, 0)),
        ],
        out_specs=pl.BlockSpec((_N_BLK, L, D_MSA), lambda n: (n, 0, 0)),
        out_shape=jax.ShapeDtypeStruct((N, L, D_MSA), jnp.float32),
        scratch_shapes=[pltpu.VMEM((L, D_MSA), jnp.float32)],
    )(seq2d, msa3, emb_W, emb_b, emb_q)


@jax.jit
def kernel(msa, seq, idx, emb_W, emb_b, emb_q, emb_left, emb_right,
           emb_state, pos_emb):
    seq1 = seq.reshape(L).astype(jnp.int32)
    idx1 = idx.reshape(L).astype(jnp.int32)

    pair, state = _sc_pair_state(seq1, idx1, emb_left, emb_right, pos_emb,
                                 emb_state)
    msa_e = _tc_msa(seq1.reshape(L, 1), msa.reshape(N, L, D_INIT),
                    emb_W, emb_b.reshape(1, D_MSA), emb_q)

    return (msa_e.reshape(B, N, L, D_MSA),
            pair.reshape(B, L, L, D_PAIR),
            state.reshape(B, L, D_STATE))


# speedup vs baseline: 1.0255x; 1.0255x over previous
import functools
import jax
import jax.numpy as jnp
from jax import lax
from jax.experimental import pallas as pl
from jax.experimental.pallas import tpu as pltpu
from jax.experimental.pallas import tpu_sc as plsc

B, N, L = 1, 128, 384
D_INIT, D_MSA, D_PAIR, D_STATE = 48, 256, 128, 32
NBIN = 65
NSEQ = 22

_NW = 32
_ROWS_PER_W = L // _NW


def _sc_body(seq_hbm, idx_hbm, left_hbm, right_hbm, pos_hbm, sttbl_hbm,
             pair_out, state_out,
             seq_v, idx_v, left_v, right_v, pos_v, sttbl_v,
             rowbuf0, rowbuf1, stbuf, sem0, sem1):
    cid = lax.axis_index("c")
    sid = lax.axis_index("s")
    w = sid * 2 + cid
    base = w * _ROWS_PER_W

    pltpu.sync_copy(seq_hbm, seq_v)
    pltpu.sync_copy(idx_hbm, idx_v)
    pltpu.sync_copy(left_hbm, left_v)
    pltpu.sync_copy(right_hbm, right_v)
    pltpu.sync_copy(pos_hbm, pos_v)
    pltpu.sync_copy(sttbl_hbm, sttbl_v)

    iota = lax.iota(jnp.int32, 16)

    @pl.when(w < L // 16)
    def _():
        sbase = w * 16
        for jj in range(16):
            j16 = jnp.full((16,), sbase + jj, jnp.int32)
            sj = plsc.load_gather(seq_v, [j16])
            for cc in range(D_STATE // 16):
                stbuf[jj, pl.ds(cc * 16, 16)] = plsc.load_gather(
                    sttbl_v, [sj, iota + cc * 16])
        pltpu.sync_copy(stbuf, state_out.at[pl.ds(sbase, 16)])

    bufs = (rowbuf0, rowbuf1)
    sems = (sem0, sem1)
    pending = [None, None]
    for rr in range(_ROWS_PER_W):
        i = base + rr
        k = rr % 2
        if pending[k] is not None:
            pending[k].wait()
        buf = bufs[k]
        i16 = jnp.full((16,), i, jnp.int32)
        si = plsc.load_gather(seq_v, [i16])
        di = plsc.load_gather(idx_v, [i16])
        rrow = [plsc.load_gather(right_v, [si, iota + 16 * cc])
                for cc in range(D_PAIR // 16)]

        @plsc.parallel_loop(0, L, unroll=2)
        def jbody(j, buf=buf, di=di, rrow=rrow):
            j16 = jnp.full((16,), j, jnp.int32)
            sj = plsc.load_gather(seq_v, [j16])
            dj = plsc.load_gather(idx_v, [j16])
            pidx = jnp.clip(dj - di + 32, 0, NBIN - 1)
            for cc in range(D_PAIR // 16):
                lv = plsc.load_gather(left_v, [sj, iota + 16 * cc])
                pv = plsc.load_gather(pos_v, [pidx, iota + 16 * cc])
                buf[j, pl.ds(16 * cc, 16)] = lv + pv + rrow[cc]

        pending[k] = pltpu.async_copy(buf, pair_out.at[i], sems[k])
    pending[0].wait()
    pending[1].wait()


def _sc_pair_state(seq, idx, emb_left, emb_right, pos_emb, emb_state):
    mesh = plsc.VectorSubcoreMesh(core_axis_name="c", subcore_axis_name="s")
    kern = pl.kernel(
        _sc_body,
        out_type=[
            jax.ShapeDtypeStruct((L, L, D_PAIR), jnp.float32),
            jax.ShapeDtypeStruct((L, D_STATE), jnp.float32),
        ],
        mesh=mesh,
        compiler_params=pltpu.CompilerParams(needs_layout_passes=False),
        scratch_types=[
            pltpu.VMEM((L,), jnp.int32),
            pltpu.VMEM((L,), jnp.int32),
            pltpu.VMEM((NSEQ, D_PAIR), jnp.float32),
            pltpu.VMEM((NSEQ, D_PAIR), jnp.float32),
            pltpu.VMEM((NBIN, D_PAIR), jnp.float32),
            pltpu.VMEM((NSEQ, D_STATE), jnp.float32),
            pltpu.VMEM((L, D_PAIR), jnp.float32),
            pltpu.VMEM((L, D_PAIR), jnp.float32),
            pltpu.VMEM((16, D_STATE), jnp.float32),
            pltpu.SemaphoreType.DMA,
            pltpu.SemaphoreType.DMA,
        ],
    )
    return kern(seq, idx, emb_left, emb_right, pos_emb, emb_state)


_N_BLK = 8


def _tc_body(seq_ref, msa_ref, w_ref, b_ref, q_ref, out_ref, qrow):
    n = pl.program_id(0)

    @pl.when(n == 0)
    def _():
        seq = seq_ref[...]
        onehot = (seq == lax.broadcasted_iota(jnp.int32, (L, NSEQ), 1)
                  ).astype(jnp.float32)
        qrow[...] = (jnp.dot(onehot, q_ref[...],
                             preferred_element_type=jnp.float32)
                     + b_ref[...])

    x = msa_ref[...]
    y = lax.dot_general(x, w_ref[...], (((2,), (1,)), ((), ())),
                        preferred_element_type=jnp.float32)
    out_ref[...] = y + qrow[...][None]


def _tc_msa(seq2d, msa3, emb_W, emb_b, emb_q):
    grid = (N // _N_BLK,)
    return pl.pallas_call(
        _tc_body,
        grid=grid,
        in_specs=[
            pl.BlockSpec((L, 1), lambda n: (0, 0)),
            pl.BlockSpec((_N_BLK, L, D_INIT), lambda n: (n, 0, 0)),
            pl.BlockSpec((D_MSA, D_INIT), lambda n: (0, 0)),
            pl.BlockSpec((1, D_MSA), lambda n: (0, 0)),
            pl.BlockSpec((NSEQ, D_MSA), lambda n: (0, 0)),
        ],
        out_specs=pl.BlockSpec((_N_BLK, L, D_MSA), lambda n: (n, 0, 0)),
        out_shape=jax.ShapeDtypeStruct((N, L, D_MSA), jnp.float32),
        scratch_shapes=[pltpu.VMEM((L, D_MSA), jnp.float32)],
    )(seq2d, msa3, emb_W, emb_b, emb_q)


@jax.jit
def kernel(msa, seq, idx, emb_W, emb_b, emb_q, emb_left, emb_right,
           emb_state, pos_emb):
    seq1 = seq.reshape(L).astype(jnp.int32)
    idx1 = idx.reshape(L).astype(jnp.int32)

    msa_e = _tc_msa(seq1.reshape(L, 1), msa.reshape(N, L, D_INIT),
                    emb_W, emb_b.reshape(1, D_MSA), emb_q)
    pair, state = _sc_pair_state(seq1, idx1, emb_left, emb_right, pos_emb,
                                 emb_state)

    return (msa_e.reshape(B, N, L, D_MSA),
            pair.reshape(B, L, L, D_PAIR),
            state.reshape(B, L, D_STATE))


# speedup vs baseline: 1.0275x; 1.0020x over previous
import jax
import jax.numpy as jnp
from jax import lax
from jax.experimental import pallas as pl
from jax.experimental.pallas import tpu as pltpu
from jax.experimental.pallas import tpu_sc as plsc

B, N, L = 1, 128, 384
D_INIT, D_MSA, D_PAIR, D_STATE = 48, 256, 128, 32
NBIN = 65
NSEQ = 22

_NW = 32
_ROWS_PER_W = L // _NW


def _sc_body(seq_hbm, idx_hbm, left_hbm, right_hbm, pos_hbm, sttbl_hbm,
             pair_out, state_out,
             seq_v, idx_v, left_v, right_v, pos_v, sttbl_v,
             rowbuf0, rowbuf1, stbuf, sem0, sem1):
    cid = lax.axis_index("c")
    sid = lax.axis_index("s")
    w = sid * 2 + cid
    base = w * _ROWS_PER_W

    pltpu.sync_copy(seq_hbm, seq_v)
    pltpu.sync_copy(idx_hbm, idx_v)
    pltpu.sync_copy(left_hbm, left_v)
    pltpu.sync_copy(right_hbm, right_v)
    pltpu.sync_copy(pos_hbm, pos_v)
    pltpu.sync_copy(sttbl_hbm, sttbl_v)

    iota = lax.iota(jnp.int32, 16)

    @pl.when(w < L // 16)
    def _():
        sbase = w * 16
        for jj in range(16):
            j16 = jnp.full((16,), sbase + jj, jnp.int32)
            sj = plsc.load_gather(seq_v, [j16]) * D_STATE
            for cc in range(D_STATE // 16):
                stbuf[pl.ds(jj * D_STATE + cc * 16, 16)] = plsc.load_gather(
                    sttbl_v, [sj + iota + cc * 16])
        pltpu.sync_copy(stbuf, state_out.at[pl.ds(sbase * D_STATE,
                                                  16 * D_STATE)])

    bufs = (rowbuf0, rowbuf1)
    sems = (sem0, sem1)
    pending = [None, None]
    for rr in range(_ROWS_PER_W):
        i = base + rr
        k = rr % 2
        if pending[k] is not None:
            pending[k].wait()
        buf = bufs[k]
        i16 = jnp.full((16,), i, jnp.int32)
        si = plsc.load_gather(seq_v, [i16]) * D_PAIR
        di = plsc.load_gather(idx_v, [i16])
        rrow = [plsc.load_gather(right_v, [si + iota + 16 * cc])
                for cc in range(D_PAIR // 16)]

        @plsc.parallel_loop(0, L, unroll=2)
        def jbody(j, buf=buf, di=di, rrow=rrow):
            j16 = jnp.full((16,), j, jnp.int32)
            sj = plsc.load_gather(seq_v, [j16]) * D_PAIR
            dj = plsc.load_gather(idx_v, [j16])
            pidx = jnp.clip(dj - di + 32, 0, NBIN - 1) * D_PAIR
            for cc in range(D_PAIR // 16):
                lv = plsc.load_gather(left_v, [sj + iota + 16 * cc])
                pv = plsc.load_gather(pos_v, [pidx + iota + 16 * cc])
                buf[j, pl.ds(16 * cc, 16)] = lv + pv + rrow[cc]

        pending[k] = pltpu.async_copy(buf, pair_out.at[i], sems[k])
    pending[0].wait()
    pending[1].wait()


def _sc_pair_state(seq, idx, emb_left, emb_right, pos_emb, emb_state):
    mesh = plsc.VectorSubcoreMesh(core_axis_name="c", subcore_axis_name="s")
    kern = pl.kernel(
        _sc_body,
        out_type=[
            jax.ShapeDtypeStruct((L, L, D_PAIR), jnp.float32),
            jax.ShapeDtypeStruct((L * D_STATE,), jnp.float32),
        ],
        mesh=mesh,
        compiler_params=pltpu.CompilerParams(needs_layout_passes=False),
        scratch_types=[
            pltpu.VMEM((L,), jnp.int32),
            pltpu.VMEM((L,), jnp.int32),
            pltpu.VMEM((NSEQ * D_PAIR,), jnp.float32),
            pltpu.VMEM((NSEQ * D_PAIR,), jnp.float32),
            pltpu.VMEM((NBIN * D_PAIR,), jnp.float32),
            pltpu.VMEM((NSEQ * D_STATE,), jnp.float32),
            pltpu.VMEM((L, D_PAIR), jnp.float32),
            pltpu.VMEM((L, D_PAIR), jnp.float32),
            pltpu.VMEM((16 * D_STATE,), jnp.float32),
            pltpu.SemaphoreType.DMA,
            pltpu.SemaphoreType.DMA,
        ],
    )
    return kern(seq, idx, emb_left.reshape(-1), emb_right.reshape(-1),
                pos_emb.reshape(-1), emb_state.reshape(-1))


_N_BLK = 8


def _tc_body(seq_ref, msa_ref, w_ref, b_ref, q_ref, out_ref, qrow):
    n = pl.program_id(0)

    @pl.when(n == 0)
    def _():
        seq = seq_ref[...]
        onehot = (seq == lax.broadcasted_iota(jnp.int32, (L, NSEQ), 1)
                  ).astype(jnp.float32)
        qrow[...] = (jnp.dot(onehot, q_ref[...],
                             preferred_element_type=jnp.float32)
                     + b_ref[...])

    x = msa_ref[...]
    y = lax.dot_general(x, w_ref[...], (((2,), (1,)), ((), ())),
                        preferred_element_type=jnp.float32)
    out_ref[...] = y + qrow[...][None]


def _tc_msa(seq2d, msa3, emb_W, emb_b, emb_q):
    grid = (N // _N_BLK,)
    return pl.pallas_call(
        _tc_body,
        grid=grid,
        in_specs=[
            pl.BlockSpec((L, 1), lambda n: (0, 0)),
            pl.BlockSpec((_N_BLK, L, D_INIT), lambda n: (n, 0, 0)),
            pl.BlockSpec((D_MSA, D_INIT), lambda n: (0, 0)),
            pl.BlockSpec((1, D_MSA), lambda n: (0, 0)),
            pl.BlockSpec((NSEQ, D_MSA), lambda n: (0, 0)),
        ],
        out_specs=pl.BlockSpec((_N_BLK, L, D_MSA), lambda n: (n, 0, 0)),
        out_shape=jax.ShapeDtypeStruct((N, L, D_MSA), jnp.float32),
        scratch_shapes=[pltpu.VMEM((L, D_MSA), jnp.float32)],
    )(seq2d, msa3, emb_W, emb_b, emb_q)


@jax.jit
def kernel(msa, seq, idx, emb_W, emb_b, emb_q, emb_left, emb_right,
           emb_state, pos_emb):
    seq1 = seq.reshape(L).astype(jnp.int32)
    idx1 = idx.reshape(L).astype(jnp.int32)

    msa_e = _tc_msa(seq1.reshape(L, 1), msa.reshape(N, L, D_INIT),
                    emb_W, emb_b.reshape(1, D_MSA), emb_q)
    pair, state = _sc_pair_state(seq1, idx1, emb_left, emb_right, pos_emb,
                                 emb_state)

    return (msa_e.reshape(B, N, L, D_MSA),
            pair.reshape(B, L, L, D_PAIR),
            state.reshape(B, L, D_STATE))


# speedup vs baseline: 1.0941x; 1.0648x over previous
import jax
import jax.numpy as jnp
from jax import lax
from jax.experimental import pallas as pl
from jax.experimental.pallas import tpu as pltpu
from jax.experimental.pallas import tpu_sc as plsc

B, N, L = 1, 128, 384
D_INIT, D_MSA, D_PAIR, D_STATE = 48, 256, 128, 32
NBIN = 65
NSEQ = 22

_NW = 32
_ROWS_PER_W = L // _NW


_PACKED_ROW = D_PAIR // 2


def _pack_tbl(t):
    r = t.shape[0]
    tb = t.reshape(r, D_PAIR // 32, 2, 16)
    bits = lax.bitcast_convert_type(tb.astype(jnp.bfloat16), jnp.uint16
                                    ).astype(jnp.uint32)
    w = bits[:, :, 0, :] | (bits[:, :, 1, :] << 16)
    return lax.bitcast_convert_type(w, jnp.float32).reshape(-1)


def _sc_body(seq_hbm, idx_hbm, left_hbm, right_hbm, pos_hbm, sttbl_hbm,
             pair_out, state_out,
             seq_v, idx_v, left_v, right_v, pos_v, sttbl_v,
             rowbuf0, rowbuf1, stbuf, sem0, sem1):
    cid = lax.axis_index("c")
    sid = lax.axis_index("s")
    w = sid * 2 + cid
    base = w * _ROWS_PER_W

    pltpu.sync_copy(seq_hbm, seq_v)
    pltpu.sync_copy(idx_hbm, idx_v)
    pltpu.sync_copy(left_hbm, left_v)
    pltpu.sync_copy(right_hbm, right_v)
    pltpu.sync_copy(pos_hbm, pos_v)
    pltpu.sync_copy(sttbl_hbm, sttbl_v)

    iota = lax.iota(jnp.int32, 16)

    @pl.when(w < L // 16)
    def _():
        sbase = w * 16
        for jj in range(16):
            j16 = jnp.full((16,), sbase + jj, jnp.int32)
            sj = plsc.load_gather(seq_v, [j16]) * D_STATE
            for cc in range(D_STATE // 16):
                stbuf[pl.ds(jj * D_STATE + cc * 16, 16)] = plsc.load_gather(
                    sttbl_v, [sj + iota + cc * 16])
        pltpu.sync_copy(stbuf, state_out.at[pl.ds(sbase * D_STATE,
                                                  16 * D_STATE)])

    bufs = (rowbuf0, rowbuf1)
    sems = (sem0, sem1)
    pending = [None, None]
    for rr in range(_ROWS_PER_W):
        i = base + rr
        k = rr % 2
        if pending[k] is not None:
            pending[k].wait()
        buf = bufs[k]
        i16 = jnp.full((16,), i, jnp.int32)
        si = plsc.load_gather(seq_v, [i16]) * _PACKED_ROW
        di = plsc.load_gather(idx_v, [i16])
        rrow = []
        for cc in range(D_PAIR // 32):
            w = plsc.load_gather(right_v, [si + iota + 16 * cc])
            ra, rb = plsc.unpack(plsc.bitcast(w, jnp.bfloat16),
                                 format=plsc.PackFormat.INTERLEAVED)
            rrow += [ra, rb]

        @plsc.parallel_loop(0, L, unroll=2)
        def jbody(j, buf=buf, di=di, rrow=rrow):
            j16 = jnp.full((16,), j, jnp.int32)
            sj = plsc.load_gather(seq_v, [j16]) * _PACKED_ROW
            dj = plsc.load_gather(idx_v, [j16])
            pidx = jnp.clip(dj - di + 32, 0, NBIN - 1) * _PACKED_ROW
            for cc in range(D_PAIR // 32):
                lw = plsc.load_gather(left_v, [sj + iota + 16 * cc])
                pw = plsc.load_gather(pos_v, [pidx + iota + 16 * cc])
                la, lb = plsc.unpack(plsc.bitcast(lw, jnp.bfloat16),
                                     format=plsc.PackFormat.INTERLEAVED)
                pa, pb = plsc.unpack(plsc.bitcast(pw, jnp.bfloat16),
                                     format=plsc.PackFormat.INTERLEAVED)
                buf[j, pl.ds(32 * cc, 16)] = la + pa + rrow[2 * cc]
                buf[j, pl.ds(32 * cc + 16, 16)] = lb + pb + rrow[2 * cc + 1]

        pending[k] = pltpu.async_copy(buf, pair_out.at[i], sems[k])
    pending[0].wait()
    pending[1].wait()


def _sc_pair_state(seq, idx, emb_left, emb_right, pos_emb, emb_state):
    mesh = plsc.VectorSubcoreMesh(core_axis_name="c", subcore_axis_name="s")
    kern = pl.kernel(
        _sc_body,
        out_type=[
            jax.ShapeDtypeStruct((L, L, D_PAIR), jnp.float32),
            jax.ShapeDtypeStruct((L * D_STATE,), jnp.float32),
        ],
        mesh=mesh,
        compiler_params=pltpu.CompilerParams(needs_layout_passes=False),
        scratch_types=[
            pltpu.VMEM((L,), jnp.int32),
            pltpu.VMEM((L,), jnp.int32),
            pltpu.VMEM((NSEQ * _PACKED_ROW,), jnp.float32),
            pltpu.VMEM((NSEQ * _PACKED_ROW,), jnp.float32),
            pltpu.VMEM((66 * _PACKED_ROW,), jnp.float32),
            pltpu.VMEM((768,), jnp.float32),
            pltpu.VMEM((L, D_PAIR), jnp.float32),
            pltpu.VMEM((L, D_PAIR), jnp.float32),
            pltpu.VMEM((16 * D_STATE,), jnp.float32),
            pltpu.SemaphoreType.DMA,
            pltpu.SemaphoreType.DMA,
        ],
    )
    sttbl = jnp.zeros((768,), jnp.float32).at[:NSEQ * D_STATE].set(
        emb_state.reshape(-1))
    return kern(seq, idx, _pack_tbl(emb_left), _pack_tbl(emb_right),
                _pack_tbl(jnp.concatenate([pos_emb,
                                           jnp.zeros((1, D_PAIR),
                                                     jnp.float32)])),
                sttbl)


_N_BLK = 8


def _tc_body(seq_ref, msa_ref, w_ref, b_ref, q_ref, out_ref, qrow):
    n = pl.program_id(0)

    @pl.when(n == 0)
    def _():
        seq = seq_ref[...]
        onehot = (seq == lax.broadcasted_iota(jnp.int32, (L, NSEQ), 1)
                  ).astype(jnp.float32)
        qrow[...] = (jnp.dot(onehot, q_ref[...],
                             preferred_element_type=jnp.float32)
                     + b_ref[...])

    x = msa_ref[...]
    y = lax.dot_general(x, w_ref[...], (((2,), (1,)), ((), ())),
                        preferred_element_type=jnp.float32)
    out_ref[...] = y + qrow[...][None]


def _tc_msa(seq2d, msa3, emb_W, emb_b, emb_q):
    grid = (N // _N_BLK,)
    return pl.pallas_call(
        _tc_body,
        grid=grid,
        in_specs=[
            pl.BlockSpec((L, 1), lambda n: (0, 0)),
            pl.BlockSpec((_N_BLK, L, D_INIT), lambda n: (n, 0, 0)),
            pl.BlockSpec((D_MSA, D_INIT), lambda n: (0, 0)),
            pl.BlockSpec((1, D_MSA), lambda n: (0, 0)),
            pl.BlockSpec((NSEQ, D_MSA), lambda n: (0, 0)),
        ],
        out_specs=pl.BlockSpec((_N_BLK, L, D_MSA), lambda n: (n, 0, 0)),
        out_shape=jax.ShapeDtypeStruct((N, L, D_MSA), jnp.float32),
        scratch_shapes=[pltpu.VMEM((L, D_MSA), jnp.float32)],
    )(seq2d, msa3, emb_W, emb_b, emb_q)


@jax.jit
def kernel(msa, seq, idx, emb_W, emb_b, emb_q, emb_left, emb_right,
           emb_state, pos_emb):
    seq1 = seq.reshape(L).astype(jnp.int32)
    idx1 = idx.reshape(L).astype(jnp.int32)

    msa_e = _tc_msa(seq1.reshape(L, 1), msa.reshape(N, L, D_INIT),
                    emb_W, emb_b.reshape(1, D_MSA), emb_q)
    pair, state = _sc_pair_state(seq1, idx1, emb_left, emb_right, pos_emb,
                                 emb_state)

    return (msa_e.reshape(B, N, L, D_MSA),
            pair.reshape(B, L, L, D_PAIR),
            state.reshape(B, L, D_STATE))


# speedup vs baseline: 1.1637x; 1.0637x over previous
import jax
import jax.numpy as jnp
from jax import lax
from jax.experimental import pallas as pl
from jax.experimental.pallas import tpu as pltpu
from jax.experimental.pallas import tpu_sc as plsc

B, N, L = 1, 128, 384
D_INIT, D_MSA, D_PAIR, D_STATE = 48, 256, 128, 32
NBIN = 65
NSEQ = 22

_NW = 32
_ROWS_PER_W = L // _NW


_PACKED_ROW = D_PAIR // 2


def _pack_tbl(t):
    r = t.shape[0]
    tb = t.reshape(r, D_PAIR // 32, 2, 16)
    bits = lax.bitcast_convert_type(tb.astype(jnp.bfloat16), jnp.uint16
                                    ).astype(jnp.uint32)
    w = bits[:, :, 0, :] | (bits[:, :, 1, :] << 16)
    return lax.bitcast_convert_type(w, jnp.float32).reshape(-1)


def _sc_body(seq_hbm, idx_hbm, left_hbm, right_hbm, pos_hbm, sttbl_hbm,
             pair_out, state_out,
             seq_v, idx_v, left_v, right_v, pos_v, sttbl_v,
             rowbuf0, rowbuf1, stbuf, sem0, sem1):
    cid = lax.axis_index("c")
    sid = lax.axis_index("s")
    w = sid * 2 + cid
    base = w * _ROWS_PER_W

    pltpu.sync_copy(seq_hbm, seq_v)
    pltpu.sync_copy(idx_hbm, idx_v)
    pltpu.sync_copy(left_hbm, left_v)
    pltpu.sync_copy(right_hbm, right_v)
    pltpu.sync_copy(pos_hbm, pos_v)
    pltpu.sync_copy(sttbl_hbm, sttbl_v)

    iota = lax.iota(jnp.int32, 16)

    @pl.when(w < L // 16)
    def _():
        sbase = w * 16
        for jj in range(16):
            j16 = jnp.full((16,), sbase + jj, jnp.int32)
            sj = plsc.load_gather(seq_v, [j16]) * D_STATE
            for cc in range(D_STATE // 16):
                stbuf[pl.ds(jj * D_STATE + cc * 16, 16)] = plsc.load_gather(
                    sttbl_v, [sj + iota + cc * 16])
        pltpu.sync_copy(stbuf, state_out.at[pl.ds(sbase * D_STATE,
                                                  16 * D_STATE)])

    bufs = (rowbuf0, rowbuf1)
    sems = (sem0, sem1)
    pending = [None, None]
    for rr in range(_ROWS_PER_W):
        i = base + rr
        k = rr % 2
        if pending[k] is not None:
            pending[k].wait()
        buf = bufs[k]
        i16 = jnp.full((16,), i, jnp.int32)
        si = plsc.load_gather(seq_v, [i16]) * _PACKED_ROW
        di = plsc.load_gather(idx_v, [i16])
        rrow = []
        for cc in range(D_PAIR // 32):
            w = plsc.load_gather(right_v, [si + iota + 16 * cc])
            ra, rb = plsc.unpack(plsc.bitcast(w, jnp.bfloat16),
                                 format=plsc.PackFormat.INTERLEAVED)
            rrow += [ra, rb]

        @plsc.parallel_loop(0, L, unroll=2)
        def jbody(j, buf=buf, di=di, rrow=rrow):
            j16 = jnp.full((16,), j, jnp.int32)
            sj = plsc.load_gather(seq_v, [j16]) * _PACKED_ROW
            dj = plsc.load_gather(idx_v, [j16])
            pidx = jnp.clip(dj - di + 32, 0, NBIN - 1) * _PACKED_ROW
            for cc in range(D_PAIR // 32):
                lw = plsc.load_gather(left_v, [sj + iota + 16 * cc])
                pw = plsc.load_gather(pos_v, [pidx + iota + 16 * cc])
                lp = (plsc.bitcast(lw, jnp.bfloat16)
                      + plsc.bitcast(pw, jnp.bfloat16))
                a, b = plsc.unpack(lp, format=plsc.PackFormat.INTERLEAVED)
                buf[j, pl.ds(32 * cc, 16)] = a + rrow[2 * cc]
                buf[j, pl.ds(32 * cc + 16, 16)] = b + rrow[2 * cc + 1]

        pending[k] = pltpu.async_copy(buf, pair_out.at[i], sems[k])
    pending[0].wait()
    pending[1].wait()


def _sc_pair_state(seq, idx, emb_left, emb_right, pos_emb, emb_state):
    mesh = plsc.VectorSubcoreMesh(core_axis_name="c", subcore_axis_name="s")
    kern = pl.kernel(
        _sc_body,
        out_type=[
            jax.ShapeDtypeStruct((L, L, D_PAIR), jnp.float32),
            jax.ShapeDtypeStruct((L * D_STATE,), jnp.float32),
        ],
        mesh=mesh,
        compiler_params=pltpu.CompilerParams(needs_layout_passes=False),
        scratch_types=[
            pltpu.VMEM((L,), jnp.int32),
            pltpu.VMEM((L,), jnp.int32),
            pltpu.VMEM((NSEQ * _PACKED_ROW,), jnp.float32),
            pltpu.VMEM((NSEQ * _PACKED_ROW,), jnp.float32),
            pltpu.VMEM((66 * _PACKED_ROW,), jnp.float32),
            pltpu.VMEM((768,), jnp.float32),
            pltpu.VMEM((L, D_PAIR), jnp.float32),
            pltpu.VMEM((L, D_PAIR), jnp.float32),
            pltpu.VMEM((16 * D_STATE,), jnp.float32),
            pltpu.SemaphoreType.DMA,
            pltpu.SemaphoreType.DMA,
        ],
    )
    sttbl = jnp.zeros((768,), jnp.float32).at[:NSEQ * D_STATE].set(
        emb_state.reshape(-1))
    return kern(seq, idx, _pack_tbl(emb_left), _pack_tbl(emb_right),
                _pack_tbl(jnp.concatenate([pos_emb,
                                           jnp.zeros((1, D_PAIR),
                                                     jnp.float32)])),
                sttbl)


_N_BLK = 8


def _tc_body(seq_ref, msa_ref, w_ref, b_ref, q_ref, out_ref, qrow):
    n = pl.program_id(0)

    @pl.when(n == 0)
    def _():
        seq = seq_ref[...]
        onehot = (seq == lax.broadcasted_iota(jnp.int32, (L, NSEQ), 1)
                  ).astype(jnp.float32)
        qrow[...] = (jnp.dot(onehot, q_ref[...],
                             preferred_element_type=jnp.float32)
                     + b_ref[...])

    x = msa_ref[...]
    y = lax.dot_general(x, w_ref[...], (((2,), (1,)), ((), ())),
                        preferred_element_type=jnp.float32)
    out_ref[...] = y + qrow[...][None]


def _tc_msa(seq2d, msa3, emb_W, emb_b, emb_q):
    grid = (N // _N_BLK,)
    return pl.pallas_call(
        _tc_body,
        grid=grid,
        in_specs=[
            pl.BlockSpec((L, 1), lambda n: (0, 0)),
            pl.BlockSpec((_N_BLK, L, D_INIT), lambda n: (n, 0, 0)),
            pl.BlockSpec((D_MSA, D_INIT), lambda n: (0, 0)),
            pl.BlockSpec((1, D_MSA), lambda n: (0, 0)),
            pl.BlockSpec((NSEQ, D_MSA), lambda n: (0, 0)),
        ],
        out_specs=pl.BlockSpec((_N_BLK, L, D_MSA), lambda n: (n, 0, 0)),
        out_shape=jax.ShapeDtypeStruct((N, L, D_MSA), jnp.float32),
        scratch_shapes=[pltpu.VMEM((L, D_MSA), jnp.float32)],
    )(seq2d, msa3, emb_W, emb_b, emb_q)


@jax.jit
def kernel(msa, seq, idx, emb_W, emb_b, emb_q, emb_left, emb_right,
           emb_state, pos_emb):
    seq1 = seq.reshape(L).astype(jnp.int32)
    idx1 = idx.reshape(L).astype(jnp.int32)

    msa_e = _tc_msa(seq1.reshape(L, 1), msa.reshape(N, L, D_INIT),
                    emb_W, emb_b.reshape(1, D_MSA), emb_q)
    pair, state = _sc_pair_state(seq1, idx1, emb_left, emb_right, pos_emb,
                                 emb_state)

    return (msa_e.reshape(B, N, L, D_MSA),
            pair.reshape(B, L, L, D_PAIR),
            state.reshape(B, L, D_STATE))


# speedup vs baseline: 1.3116x; 1.1271x over previous
import jax
import jax.numpy as jnp
from jax import lax
from jax.experimental import pallas as pl
from jax.experimental.pallas import tpu as pltpu
from jax.experimental.pallas import tpu_sc as plsc

B, N, L = 1, 128, 384
D_INIT, D_MSA, D_PAIR, D_STATE = 48, 256, 128, 32
NBIN = 65
NSEQ = 22

_NW = 32
_ROWS_PER_W = L // _NW


_PACKED_ROW = D_PAIR // 2


def _pack_tbl(t):
    r = t.shape[0]
    tb = t.reshape(r, D_PAIR // 32, 2, 16)
    bits = lax.bitcast_convert_type(tb.astype(jnp.bfloat16), jnp.uint16
                                    ).astype(jnp.uint32)
    w = bits[:, :, 0, :] | (bits[:, :, 1, :] << 16)
    return lax.bitcast_convert_type(w, jnp.float32).reshape(-1)


def _sc_body(seq_hbm, idx_hbm, left_hbm, right_hbm, pos_hbm, sttbl_hbm,
             pair_out, state_out,
             seq_v, idx_v, left_v, right_v, pos_v, sttbl_v,
             rowbuf0, rowbuf1, stbuf, sem0, sem1):
    cid = lax.axis_index("c")
    sid = lax.axis_index("s")
    w = sid * 2 + cid
    base = w * _ROWS_PER_W

    pltpu.sync_copy(seq_hbm, seq_v)
    pltpu.sync_copy(idx_hbm, idx_v)
    pltpu.sync_copy(left_hbm, left_v)
    pltpu.sync_copy(right_hbm, right_v)
    pltpu.sync_copy(pos_hbm, pos_v)
    pltpu.sync_copy(sttbl_hbm, sttbl_v)

    iota = lax.iota(jnp.int32, 16)

    @pl.when(w < L // 16)
    def _():
        sbase = w * 16
        for jj in range(16):
            j16 = jnp.full((16,), sbase + jj, jnp.int32)
            sj = plsc.load_gather(seq_v, [j16]) * D_STATE
            for cc in range(D_STATE // 16):
                stbuf[pl.ds(jj * D_STATE + cc * 16, 16)] = plsc.load_gather(
                    sttbl_v, [sj + iota + cc * 16])
        pltpu.sync_copy(stbuf, state_out.at[pl.ds(sbase * D_STATE,
                                                  16 * D_STATE)])

    bufs = (rowbuf0, rowbuf1)
    sems = (sem0, sem1)
    pending = [None, None]
    for rr in range(_ROWS_PER_W):
        i = base + rr
        k = rr % 2
        if pending[k] is not None:
            pending[k].wait()
        buf = bufs[k]
        i16 = jnp.full((16,), i, jnp.int32)
        si = plsc.load_gather(seq_v, [i16]) * _PACKED_ROW
        di = plsc.load_gather(idx_v, [i16])
        rrow = []
        for cc in range(D_PAIR // 32):
            w = plsc.load_gather(right_v, [si + iota + 16 * cc])
            ra, rb = plsc.unpack(plsc.bitcast(w, jnp.bfloat16),
                                 format=plsc.PackFormat.INTERLEAVED)
            rrow += [ra, rb]

        @plsc.parallel_loop(0, L, unroll=2)
        def jbody(j, buf=buf, di=di, rrow=rrow):
            j16 = jnp.full((16,), j, jnp.int32)
            sj = plsc.load_gather(seq_v, [j16]) * _PACKED_ROW
            dj = plsc.load_gather(idx_v, [j16])
            pidx = jnp.clip(dj - di + 32, 0, NBIN - 1) * _PACKED_ROW
            for cc in range(D_PAIR // 32):
                lw = plsc.load_gather(left_v, [sj + iota + 16 * cc])
                pw = plsc.load_gather(pos_v, [pidx + iota + 16 * cc])
                lp = (plsc.bitcast(lw, jnp.bfloat16)
                      + plsc.bitcast(pw, jnp.bfloat16))
                a, b = plsc.unpack(lp, format=plsc.PackFormat.INTERLEAVED)
                buf[j, pl.ds(32 * cc, 16)] = a + rrow[2 * cc]
                buf[j, pl.ds(32 * cc + 16, 16)] = b + rrow[2 * cc + 1]

        pending[k] = pltpu.async_copy(buf, pair_out.at[i], sems[k])
    pending[0].wait()
    pending[1].wait()


def _sc_pair_state(seq, idx, emb_left, emb_right, pos_emb, emb_state):
    mesh = plsc.VectorSubcoreMesh(core_axis_name="c", subcore_axis_name="s")
    kern = pl.kernel(
        _sc_body,
        out_type=[
            jax.ShapeDtypeStruct((L, L, D_PAIR), jnp.float32),
            jax.ShapeDtypeStruct((L * D_STATE,), jnp.float32),
        ],
        mesh=mesh,
        compiler_params=pltpu.CompilerParams(needs_layout_passes=False),
        scratch_types=[
            pltpu.VMEM((L,), jnp.int32),
            pltpu.VMEM((L,), jnp.int32),
            pltpu.VMEM((NSEQ * _PACKED_ROW,), jnp.float32),
            pltpu.VMEM((NSEQ * _PACKED_ROW,), jnp.float32),
            pltpu.VMEM((66 * _PACKED_ROW,), jnp.float32),
            pltpu.VMEM((768,), jnp.float32),
            pltpu.VMEM((L, D_PAIR), jnp.float32),
            pltpu.VMEM((L, D_PAIR), jnp.float32),
            pltpu.VMEM((16 * D_STATE,), jnp.float32),
            pltpu.SemaphoreType.DMA,
            pltpu.SemaphoreType.DMA,
        ],
    )
    sttbl = jnp.zeros((768,), jnp.float32).at[:NSEQ * D_STATE].set(
        emb_state.reshape(-1))
    return kern(seq, idx, _pack_tbl(emb_left), _pack_tbl(emb_right),
                _pack_tbl(jnp.concatenate([pos_emb,
                                           jnp.zeros((1, D_PAIR),
                                                     jnp.float32)])),
                sttbl)


_N_BLK = 8


def _tc_body(seq_ref, msa_ref, w_ref, b_ref, q_ref, out_ref, qrow):
    n = pl.program_id(0)

    @pl.when(n == 0)
    def _():
        seq = seq_ref[...]
        onehot = (seq == lax.broadcasted_iota(jnp.int32, (L, NSEQ), 1)
                  ).astype(jnp.float32)
        qrow[...] = (jnp.dot(onehot, q_ref[...],
                             preferred_element_type=jnp.float32)
                     + b_ref[...])

    for b in range(_N_BLK):
        x = msa_ref[b]
        y = lax.dot_general(x, w_ref[...], (((0,), (0,)), ((), ())),
                            preferred_element_type=jnp.float32)
        out_ref[b] = y + qrow[...]


def _tc_msa(seq2d, msa3t, emb_Wt, emb_b, emb_q):
    grid = (N // _N_BLK,)
    return pl.pallas_call(
        _tc_body,
        grid=grid,
        in_specs=[
            pl.BlockSpec((L, 1), lambda n: (0, 0)),
            pl.BlockSpec((_N_BLK, D_INIT, L), lambda n: (n, 0, 0)),
            pl.BlockSpec((D_INIT, D_MSA), lambda n: (0, 0)),
            pl.BlockSpec((1, D_MSA), lambda n: (0, 0)),
            pl.BlockSpec((NSEQ, D_MSA), lambda n: (0, 0)),
        ],
        out_specs=pl.BlockSpec((_N_BLK, L, D_MSA), lambda n: (n, 0, 0)),
        out_shape=jax.ShapeDtypeStruct((N, L, D_MSA), jnp.float32),
        scratch_shapes=[pltpu.VMEM((L, D_MSA), jnp.float32)],
    )(seq2d, msa3t, emb_Wt, emb_b, emb_q)


@jax.jit
def kernel(msa, seq, idx, emb_W, emb_b, emb_q, emb_left, emb_right,
           emb_state, pos_emb):
    seq1 = seq.reshape(L).astype(jnp.int32)
    idx1 = idx.reshape(L).astype(jnp.int32)

    msa_e = _tc_msa(seq1.reshape(L, 1),
                    msa.reshape(N, L, D_INIT).transpose(0, 2, 1),
                    emb_W.T, emb_b.reshape(1, D_MSA), emb_q)
    pair, state = _sc_pair_state(seq1, idx1, emb_left, emb_right, pos_emb,
                                 emb_state)

    return (msa_e.reshape(B, N, L, D_MSA),
            pair.reshape(B, L, L, D_PAIR),
            state.reshape(B, L, D_STATE))


# speedup vs baseline: 1.3800x; 1.0521x over previous
import jax
import jax.numpy as jnp
from jax import lax
from jax.experimental import pallas as pl
from jax.experimental.pallas import tpu as pltpu
from jax.experimental.pallas import tpu_sc as plsc

B, N, L = 1, 128, 384
D_INIT, D_MSA, D_PAIR, D_STATE = 48, 256, 128, 32
NBIN = 65
NSEQ = 22

_NW = 32
_ROWS_PER_W = L // _NW


_PACKED_ROW = D_PAIR // 2
_RIGHT_OFF = NSEQ * _PACKED_ROW
_POS_OFF = 2 * NSEQ * _PACKED_ROW
_TBL_WORDS = _POS_OFF + 66 * _PACKED_ROW


def _pack_tbl(t):
    r = t.shape[0]
    tb = t.reshape(r, D_PAIR // 32, 2, 16)
    bits = lax.bitcast_convert_type(tb.astype(jnp.bfloat16), jnp.uint16
                                    ).astype(jnp.uint32)
    w = bits[:, :, 0, :] | (bits[:, :, 1, :] << 16)
    return lax.bitcast_convert_type(w, jnp.float32).reshape(-1)


def _sc_body(seq_hbm, idx_hbm, tbl_hbm, sttbl_hbm,
             pair_out, state_out,
             seq_v, idx_v, tbl_v, sttbl_v, combo_v,
             rowbuf0, rowbuf1, stbuf, sem0, sem1):
    cid = lax.axis_index("c")
    sid = lax.axis_index("s")
    w = sid * 2 + cid
    base = w * _ROWS_PER_W

    pltpu.sync_copy(seq_hbm, seq_v)
    pltpu.sync_copy(idx_hbm, idx_v)
    pltpu.sync_copy(tbl_hbm, tbl_v)
    pltpu.sync_copy(sttbl_hbm, sttbl_v)

    iota = lax.iota(jnp.int32, 16)

    @pl.when(w < L // 16)
    def _():
        sbase = w * 16
        for jj in range(16):
            j16 = jnp.full((16,), sbase + jj, jnp.int32)
            sj = plsc.load_gather(seq_v, [j16]) * D_STATE
            for cc in range(D_STATE // 16):
                stbuf[pl.ds(jj * D_STATE + cc * 16, 16)] = plsc.load_gather(
                    sttbl_v, [sj + iota + cc * 16])
        pltpu.sync_copy(stbuf, state_out.at[pl.ds(sbase * D_STATE,
                                                  16 * D_STATE)])

    bufs = (rowbuf0, rowbuf1)
    sems = (sem0, sem1)
    pending = [None, None]
    for rr in range(_ROWS_PER_W):
        i = base + rr
        k = rr % 2
        if pending[k] is not None:
            pending[k].wait()
        buf = bufs[k]
        i16 = jnp.full((16,), i, jnp.int32)
        si = plsc.load_gather(seq_v, [i16]) * _PACKED_ROW
        di = plsc.load_gather(idx_v, [i16])
        rrow = []
        for cc in range(D_PAIR // 32):
            rw = plsc.load_gather(tbl_v, [_RIGHT_OFF + si + iota + 16 * cc])
            ra, rb = plsc.unpack(plsc.bitcast(rw, jnp.bfloat16),
                                 format=plsc.PackFormat.INTERLEAVED)
            rrow += [ra, rb]

        @plsc.parallel_loop(0, L // 16, unroll=2)
        def cbody(jb, di=di):
            sjv = seq_v[pl.ds(jb * 16, 16)] * _PACKED_ROW
            djv = idx_v[pl.ds(jb * 16, 16)]
            pidx = (jnp.clip(djv - di + 32, 0, NBIN - 1) * _PACKED_ROW
                    + _POS_OFF)
            combo_v[pl.ds(jb * 16, 16)] = (pidx << 16) | sjv

        @plsc.parallel_loop(0, L, unroll=2)
        def jbody(j, buf=buf, rrow=rrow):
            j16 = jnp.full((16,), j, jnp.int32)
            cw = plsc.load_gather(combo_v, [j16])
            sj = cw & 0xFFFF
            pidx = lax.shift_right_logical(cw, 16)
            for cc in range(D_PAIR // 32):
                lw = plsc.load_gather(tbl_v, [sj + iota + 16 * cc])
                pw = plsc.load_gather(tbl_v, [pidx + iota + 16 * cc])
                lp = (plsc.bitcast(lw, jnp.bfloat16)
                      + plsc.bitcast(pw, jnp.bfloat16))
                a, b = plsc.unpack(lp, format=plsc.PackFormat.INTERLEAVED)
                buf[j, pl.ds(32 * cc, 16)] = a + rrow[2 * cc]
                buf[j, pl.ds(32 * cc + 16, 16)] = b + rrow[2 * cc + 1]

        pending[k] = pltpu.async_copy(buf, pair_out.at[i], sems[k])
    pending[0].wait()
    pending[1].wait()


def _sc_pair_state(seq, idx, emb_left, emb_right, pos_emb, emb_state):
    mesh = plsc.VectorSubcoreMesh(core_axis_name="c", subcore_axis_name="s")
    kern = pl.kernel(
        _sc_body,
        out_type=[
            jax.ShapeDtypeStruct((L, L, D_PAIR), jnp.float32),
            jax.ShapeDtypeStruct((L * D_STATE,), jnp.float32),
        ],
        mesh=mesh,
        compiler_params=pltpu.CompilerParams(needs_layout_passes=False),
        scratch_types=[
            pltpu.VMEM((L,), jnp.int32),
            pltpu.VMEM((L,), jnp.int32),
            pltpu.VMEM((_TBL_WORDS,), jnp.float32),
            pltpu.VMEM((768,), jnp.float32),
            pltpu.VMEM((L,), jnp.int32),
            pltpu.VMEM((L, D_PAIR), jnp.float32),
            pltpu.VMEM((L, D_PAIR), jnp.float32),
            pltpu.VMEM((16 * D_STATE,), jnp.float32),
            pltpu.SemaphoreType.DMA,
            pltpu.SemaphoreType.DMA,
        ],
    )
    sttbl = jnp.zeros((768,), jnp.float32).at[:NSEQ * D_STATE].set(
        emb_state.reshape(-1))
    tbl = _pack_tbl(jnp.concatenate(
        [emb_left, emb_right, pos_emb,
         jnp.zeros((1, D_PAIR), jnp.float32)]))
    return kern(seq, idx, tbl, sttbl)


_N_BLK = 8


def _tc_body(seq_ref, msa_ref, w_ref, b_ref, q_ref, out_ref, qrow):
    n = pl.program_id(0)

    @pl.when(n == 0)
    def _():
        seq = seq_ref[...]
        onehot = (seq == lax.broadcasted_iota(jnp.int32, (L, NSEQ), 1)
                  ).astype(jnp.float32)
        qrow[...] = (jnp.dot(onehot, q_ref[...],
                             preferred_element_type=jnp.float32)
                     + b_ref[...])

    for b in range(_N_BLK):
        x = msa_ref[b]
        y = lax.dot_general(x, w_ref[...], (((0,), (0,)), ((), ())),
                            preferred_element_type=jnp.float32)
        out_ref[b] = y + qrow[...]


def _tc_msa(seq2d, msa3t, emb_Wt, emb_b, emb_q):
    grid = (N // _N_BLK,)
    return pl.pallas_call(
        _tc_body,
        grid=grid,
        in_specs=[
            pl.BlockSpec((L, 1), lambda n: (0, 0)),
            pl.BlockSpec((_N_BLK, D_INIT, L), lambda n: (n, 0, 0)),
            pl.BlockSpec((D_INIT, D_MSA), lambda n: (0, 0)),
            pl.BlockSpec((1, D_MSA), lambda n: (0, 0)),
            pl.BlockSpec((NSEQ, D_MSA), lambda n: (0, 0)),
        ],
        out_specs=pl.BlockSpec((_N_BLK, L, D_MSA), lambda n: (n, 0, 0)),
        out_shape=jax.ShapeDtypeStruct((N, L, D_MSA), jnp.float32),
        scratch_shapes=[pltpu.VMEM((L, D_MSA), jnp.float32)],
    )(seq2d, msa3t, emb_Wt, emb_b, emb_q)


@jax.jit
def kernel(msa, seq, idx, emb_W, emb_b, emb_q, emb_left, emb_right,
           emb_state, pos_emb):
    seq1 = seq.reshape(L).astype(jnp.int32)
    idx1 = idx.reshape(L).astype(jnp.int32)

    msa_e = _tc_msa(seq1.reshape(L, 1),
                    msa.reshape(N, L, D_INIT).transpose(0, 2, 1),
                    emb_W.T, emb_b.reshape(1, D_MSA), emb_q)
    pair, state = _sc_pair_state(seq1, idx1, emb_left, emb_right, pos_emb,
                                 emb_state)

    return (msa_e.reshape(B, N, L, D_MSA),
            pair.reshape(B, L, L, D_PAIR),
            state.reshape(B, L, D_STATE))


# speedup vs baseline: 1.3839x; 1.0028x over previous
import jax
import jax.numpy as jnp
from jax import lax
from jax.experimental import pallas as pl
from jax.experimental.pallas import tpu as pltpu
from jax.experimental.pallas import tpu_sc as plsc

B, N, L = 1, 128, 384
D_INIT, D_MSA, D_PAIR, D_STATE = 48, 256, 128, 32
NBIN = 65
NSEQ = 22

_NW = 32
_ROWS_PER_W = L // _NW


_PACKED_ROW = D_PAIR // 2
_RIGHT_OFF = NSEQ * _PACKED_ROW
_POS_OFF = 2 * NSEQ * _PACKED_ROW
_TBL_ROWS = 2 * NSEQ + NBIN + 1
_TBL_WORDS = _TBL_ROWS * _PACKED_ROW


def _sc_body(seq_hbm, idx_hbm, rawtbl_hbm, sttbl_hbm,
             pair_out, statet_out,
             seq_v, idx_v, raw_v, tbl_v, sttbl_v, combo_v,
             rowbuf0, rowbuf1, stbuf_t, sem0, sem1):
    cid = lax.axis_index("c")
    sid = lax.axis_index("s")
    w = sid * 2 + cid
    base = w * _ROWS_PER_W

    pltpu.sync_copy(seq_hbm, seq_v)
    pltpu.sync_copy(idx_hbm, idx_v)
    pltpu.sync_copy(rawtbl_hbm, raw_v)
    pltpu.sync_copy(sttbl_hbm, sttbl_v)

    iota = lax.iota(jnp.int32, 16)

    @plsc.parallel_loop(0, _TBL_ROWS, unroll=2)
    def pbody(r):
        for cc in range(D_PAIR // 32):
            a = raw_v[r, pl.ds(32 * cc, 16)]
            b = raw_v[r, pl.ds(32 * cc + 16, 16)]
            pk = plsc.pack(a, b, format=plsc.PackFormat.INTERLEAVED)
            tbl_v[pl.ds(r * _PACKED_ROW + 16 * cc, 16)] = plsc.bitcast(
                pk, jnp.float32)

    @pl.when(w < 3)
    def _():
        for jb in range(8):
            sjv = seq_v[pl.ds(w * 128 + jb * 16, 16)] * D_STATE
            for f in range(D_STATE):
                stbuf_t[f, pl.ds(jb * 16, 16)] = plsc.load_gather(
                    sttbl_v, [sjv + f])
        pltpu.sync_copy(stbuf_t, statet_out.at[:, pl.ds(w * 128, 128)])

    bufs = (rowbuf0, rowbuf1)
    sems = (sem0, sem1)
    pending = [None, None]
    for rr in range(_ROWS_PER_W):
        i = base + rr
        k = rr % 2
        if pending[k] is not None:
            pending[k].wait()
        buf = bufs[k]
        i16 = jnp.full((16,), i, jnp.int32)
        si = plsc.load_gather(seq_v, [i16]) * _PACKED_ROW
        di = plsc.load_gather(idx_v, [i16])
        rrow = []
        for cc in range(D_PAIR // 32):
            rw = plsc.load_gather(tbl_v, [_RIGHT_OFF + si + iota + 16 * cc])
            ra, rb = plsc.unpack(plsc.bitcast(rw, jnp.bfloat16),
                                 format=plsc.PackFormat.INTERLEAVED)
            rrow += [ra, rb]

        @plsc.parallel_loop(0, L // 16, unroll=2)
        def cbody(jb, di=di):
            sjv = seq_v[pl.ds(jb * 16, 16)] * _PACKED_ROW
            djv = idx_v[pl.ds(jb * 16, 16)]
            pidx = (jnp.clip(djv - di + 32, 0, NBIN - 1) * _PACKED_ROW
                    + _POS_OFF)
            combo_v[pl.ds(jb * 16, 16)] = (pidx << 16) | sjv

        @plsc.parallel_loop(0, L, unroll=2)
        def jbody(j, buf=buf, rrow=rrow):
            j16 = jnp.full((16,), j, jnp.int32)
            cw = plsc.load_gather(combo_v, [j16])
            sj = cw & 0xFFFF
            pidx = lax.shift_right_logical(cw, 16)
            for cc in range(D_PAIR // 32):
                lw = plsc.load_gather(tbl_v, [sj + iota + 16 * cc])
                pw = plsc.load_gather(tbl_v, [pidx + iota + 16 * cc])
                lp = (plsc.bitcast(lw, jnp.bfloat16)
                      + plsc.bitcast(pw, jnp.bfloat16))
                a, b = plsc.unpack(lp, format=plsc.PackFormat.INTERLEAVED)
                buf[j, pl.ds(32 * cc, 16)] = a + rrow[2 * cc]
                buf[j, pl.ds(32 * cc + 16, 16)] = b + rrow[2 * cc + 1]

        pending[k] = pltpu.async_copy(buf, pair_out.at[i], sems[k])
    pending[0].wait()
    pending[1].wait()


def _sc_pair_state(seq, idx, emb_left, emb_right, pos_emb, emb_state):
    mesh = plsc.VectorSubcoreMesh(core_axis_name="c", subcore_axis_name="s")
    kern = pl.kernel(
        _sc_body,
        out_type=[
            jax.ShapeDtypeStruct((L, L, D_PAIR), jnp.float32),
            jax.ShapeDtypeStruct((D_STATE, L), jnp.float32),
        ],
        mesh=mesh,
        compiler_params=pltpu.CompilerParams(needs_layout_passes=False),
        scratch_types=[
            pltpu.VMEM((L,), jnp.int32),
            pltpu.VMEM((L,), jnp.int32),
            pltpu.VMEM((_TBL_ROWS, D_PAIR), jnp.float32),
            pltpu.VMEM((_TBL_WORDS,), jnp.float32),
            pltpu.VMEM((NSEQ * D_STATE,), jnp.float32),
            pltpu.VMEM((L,), jnp.int32),
            pltpu.VMEM((L, D_PAIR), jnp.float32),
            pltpu.VMEM((L, D_PAIR), jnp.float32),
            pltpu.VMEM((D_STATE, 128), jnp.float32),
            pltpu.SemaphoreType.DMA,
            pltpu.SemaphoreType.DMA,
        ],
    )
    rawtbl = jnp.concatenate([emb_left, emb_right, pos_emb,
                              jnp.zeros((1, D_PAIR), jnp.float32)])
    return kern(seq, idx, rawtbl, emb_state.reshape(-1))


_N_BLK = 8


def _tc_body(seq_ref, msa_ref, w_ref, b_ref, q_ref, out_ref, qrow):
    n = pl.program_id(0)

    @pl.when(n == 0)
    def _():
        seq = seq_ref[...]
        onehot_t = (jnp.broadcast_to(seq, (NSEQ, L))
                    == lax.broadcasted_iota(jnp.int32, (NSEQ, L), 0)
                    ).astype(jnp.float32)
        qrow[...] = (lax.dot_general(onehot_t, q_ref[...],
                                     (((0,), (0,)), ((), ())),
                                     preferred_element_type=jnp.float32)
                     + b_ref[...])

    for b in range(_N_BLK):
        x = msa_ref[b]
        y = lax.dot_general(x, w_ref[...], (((0,), (0,)), ((), ())),
                            preferred_element_type=jnp.float32)
        out_ref[b] = y + qrow[...]


def _tc_msa(seq2d, msa3t, emb_Wt, emb_b, emb_q):
    grid = (N // _N_BLK,)
    return pl.pallas_call(
        _tc_body,
        grid=grid,
        in_specs=[
            pl.BlockSpec((1, L), lambda n: (0, 0)),
            pl.BlockSpec((_N_BLK, D_INIT, L), lambda n: (n, 0, 0)),
            pl.BlockSpec((D_INIT, D_MSA), lambda n: (0, 0)),
            pl.BlockSpec((1, D_MSA), lambda n: (0, 0)),
            pl.BlockSpec((NSEQ, D_MSA), lambda n: (0, 0)),
        ],
        out_specs=pl.BlockSpec((_N_BLK, L, D_MSA), lambda n: (n, 0, 0)),
        out_shape=jax.ShapeDtypeStruct((N, L, D_MSA), jnp.float32),
        scratch_shapes=[pltpu.VMEM((L, D_MSA), jnp.float32)],
    )(seq2d, msa3t, emb_Wt, emb_b, emb_q)


@jax.jit
def kernel(msa, seq, idx, emb_W, emb_b, emb_q, emb_left, emb_right,
           emb_state, pos_emb):
    seq1 = seq.reshape(L).astype(jnp.int32)
    idx1 = idx.reshape(L).astype(jnp.int32)

    msa_e = _tc_msa(seq1.reshape(1, L),
                    msa.reshape(N, L, D_INIT).transpose(0, 2, 1),
                    emb_W.T, emb_b.reshape(1, D_MSA), emb_q)
    pair, statet = _sc_pair_state(seq1, idx1, emb_left, emb_right, pos_emb,
                                  emb_state)

    return (msa_e.reshape(B, N, L, D_MSA),
            pair.reshape(B, L, L, D_PAIR),
            statet.T.reshape(B, L, D_STATE))
